# Initial kernel scaffold; baseline (speedup 1.0000x reference)
import jax
import jax.numpy as jnp
from jax.experimental import pallas as pl


def kernel(x_SB, x_PQ, x_PV, x_NB, ei_SB_PQ, ei_PQ_SB, ei_PV_PQ, ei_PQ_PV, ei_NB_PQ, ei_PQ_NB, ei_PQ_PQ, ea_SB_PQ, ea_PQ_SB, ea_PV_PQ, ea_PQ_PV, ea_NB_PQ, ea_PQ_NB, ea_PQ_PQ, params):
    raise NotImplementedError("write your pallas kernel here")


# speedup vs baseline: 5.4774x; 5.4774x over previous
import functools

import jax
import jax.numpy as jnp
from jax import lax
from jax.experimental import pallas as pl
from jax.experimental.pallas import tpu as pltpu
from jax.experimental.pallas import tpu_sc as plsc

_H = 64
_TYPES = ("SB", "PQ", "PV", "NB")
_N = {"SB": 500, "PQ": 30000, "PV": 15000, "NB": 4500}
_ETS = (("SB", "PQ"), ("PQ", "SB"), ("PV", "PQ"), ("PQ", "PV"),
        ("NB", "PQ"), ("PQ", "NB"), ("PQ", "PQ"))
_E = {("SB", "PQ"): 50000, ("PQ", "SB"): 50000, ("PV", "PQ"): 150000,
      ("PQ", "PV"): 150000, ("NB", "PQ"): 50000, ("PQ", "NB"): 50000,
      ("PQ", "PQ"): 300000}


def _rup(x, m):
    return (x + m - 1) // m * m


_NPAD = {t: _rup(_N[t] + 1, 256) for t in _TYPES}
_NACC = {t: _rup(_N[t] + 1, 8) for t in _TYPES}
_EPAD = {et: _rup(_E[et], 4096) for et in _ETS}
_DST_ETS = {t: tuple(et for et in _ETS if et[1] == t) for t in _TYPES}
_SRC_ETS = {t: tuple(et for et in _ETS if et[0] == t) for t in _TYPES}

_QU_W = 80
_KV_W = 128


def _col_layout(t):
    offs = {}
    c = 0
    for et in _DST_ETS[t]:
        offs[("qu", et)] = c
        c += _QU_W
    for et in _SRC_ETS[t]:
        offs[("kv", et)] = c
        c += _KV_W
    offs["root"] = c
    c += _H
    return offs, _rup(c, 128)


_LAYOUT = {t: _col_layout(t) for t in _TYPES}


def _mm_body(x_ref, w_ref, o_ref):
    o_ref[...] = jnp.dot(x_ref[...], w_ref[...],
                         preferred_element_type=jnp.float32)


def _mm(x, w):
    m, k = x.shape
    _, n = w.shape
    bm = 256
    return pl.pallas_call(
        _mm_body,
        grid=(m // bm,),
        in_specs=[pl.BlockSpec((bm, k), lambda i: (i, 0)),
                  pl.BlockSpec((k, n), lambda i: (0, 0))],
        out_specs=pl.BlockSpec((bm, n), lambda i: (i, 0)),
        out_shape=jax.ShapeDtypeStruct((m, n), jnp.float32),
    )(x, w)


def _merge_body(n_real, n_ets, bm, *refs):
    acc_refs = refs[:n_ets]
    ss_refs = refs[n_ets:2 * n_ets]
    root_ref = refs[2 * n_ets]
    o_ref = refs[2 * n_ets + 1]
    val = root_ref[...]
    for a_r, s_r in zip(acc_refs, ss_refs):
        a = a_r[0] + a_r[1]
        s = s_r[0] + s_r[1]
        val = val + a / (s[:, None] + 1e-16)
    neg = jnp.exp(jnp.minimum(val, 0.0)) - 1.0
    val = jnp.where(val > 0, val, neg)
    i = pl.program_id(0)
    rows = i * bm + lax.broadcasted_iota(jnp.int32, (bm, _H), 0)
    o_ref[...] = jnp.where(rows < n_real, val, 0.0)


def _merge(accs, ssums, root, n_real, npad):
    n_ets = len(accs)
    bm = 256
    in_specs = (
        [pl.BlockSpec((2, bm, _H), lambda i: (0, i, 0)) for _ in accs]
        + [pl.BlockSpec((2, bm), lambda i: (0, i)) for _ in ssums]
        + [pl.BlockSpec((bm, _H), lambda i: (i, 0))]
    )
    return pl.pallas_call(
        functools.partial(_merge_body, n_real, n_ets, bm),
        grid=(npad // bm,),
        in_specs=in_specs,
        out_specs=pl.BlockSpec((bm, _H), lambda i: (i, 0)),
        out_shape=jax.ShapeDtypeStruct((npad, _H), jnp.float32),
    )(*accs, *ssums, root)


@functools.lru_cache(None)
def _edge_kernel(epad, npad, nacc):
    chunk = epad // 32
    nblk = chunk // 128
    rpt = _rup((nacc + 15) // 16, 8)
    last = nacc - 15 * rpt
    n16_nom, tail_nom = rpt // 16, rpt % 16
    n16_last, tail_last = last // 16, last % 16
    mesh = plsc.VectorSubcoreMesh(core_axis_name="c", subcore_axis_name="s")

    @functools.partial(
        pl.kernel,
        mesh=mesh,
        compiler_params=pltpu.CompilerParams(needs_layout_passes=False,
                                             use_tc_tiling_on_sc=False),
        out_type=(jax.ShapeDtypeStruct((2, npad, _H), jnp.float32),
                  jax.ShapeDtypeStruct((2, npad), jnp.float32)),
        scratch_types=[
            pltpu.VMEM((128,), jnp.int32),
            pltpu.VMEM((128,), jnp.int32),
            pltpu.VMEM((128,), jnp.float32),
            pltpu.VMEM((128,), jnp.float32),
            pltpu.VMEM((16, _QU_W), jnp.float32),
            pltpu.VMEM((16, _KV_W), jnp.float32),
            pltpu.VMEM((16, _H), jnp.float32),
            pltpu.VMEM((16,), jnp.float32),
            pltpu.VMEM((128,), jnp.float32),
            pltpu.VMEM_SHARED((nacc, _H), jnp.float32),
            pltpu.VMEM_SHARED((nacc,), jnp.float32),
            pltpu.SemaphoreType.DMA,
            pltpu.SemaphoreType.DMA,
        ],
    )
    def k(qu_h, kv_h, src_h, dst_h, ea0_h, ea1_h, wev_h,
          acc_o, ss_o, srcb, dstb, ea0b, ea1b, qub, kvb, vst, ebuf, wevb,
          accs, sss, sem1, sem2):
        cid = lax.axis_index("c")
        sid = lax.axis_index("s")
        wid = sid * 2 + cid
        base = wid * chunk
        pltpu.sync_copy(wev_h, wevb)
        r0 = sid * rpt
        n16 = jnp.where(sid == 15, n16_last, n16_nom)
        z16 = jnp.zeros((16,), jnp.float32)
        for r in range(16):
            for t in range(4):
                vst[r, pl.ds(t * 16, 16)] = z16
        ebuf[...] = z16

        def zbody(i, c):
            pltpu.sync_copy(vst, accs.at[pl.ds(r0 + i * 16, 16)])
            pltpu.sync_copy(ebuf, sss.at[pl.ds(r0 + i * 16, 16)])
            return c

        lax.fori_loop(0, n16, zbody, 0)
        for tail, nf, pred in ((tail_nom, n16_nom, sid != 15),
                               (tail_last, n16_last, sid == 15)):
            if tail:
                @pl.when(pred)
                def _(tail=tail, nf=nf):
                    o = r0 + nf * 16
                    pltpu.sync_copy(vst.at[pl.ds(0, tail)],
                                    accs.at[pl.ds(o, tail)])
                    pltpu.sync_copy(ebuf.at[pl.ds(0, tail)],
                                    sss.at[pl.ds(o, tail)])
        plsc.subcore_barrier()

        lanes = lax.iota(jnp.int32, 16)

        def blk(b, carry):
            boff = base + b * 128
            pltpu.sync_copy(src_h.at[pl.ds(boff, 128)], srcb)
            pltpu.sync_copy(dst_h.at[pl.ds(boff, 128)], dstb)
            pltpu.sync_copy(ea0_h.at[pl.ds(boff, 128)], ea0b)
            pltpu.sync_copy(ea1_h.at[pl.ds(boff, 128)], ea1b)
            lax.fori_loop(0, 8, grp, 0)
            return carry

        def grp(g, carry):
            off = g * 16
            dst16 = dstb[pl.ds(off, 16)]
            src16 = srcb[pl.ds(off, 16)]
            cp1 = pltpu.async_copy(qu_h.at[dst16], qub, sem1)
            cp2 = pltpu.async_copy(kv_h.at[src16], kvb, sem2)
            cp1.wait()
            cp2.wait()
            ea0v = ea0b[pl.ds(off, 16)]
            ea1v = ea1b[pl.ds(off, 16)]
            logit = jnp.zeros((16,), jnp.float32)
            for j in range(16):
                prod = qub[j, pl.ds(0, 16)] * kvb[j, pl.ds(0, 16)]
                for t in range(1, 4):
                    prod = prod + (qub[j, pl.ds(t * 16, 16)]
                                   * kvb[j, pl.ds(t * 16, 16)])
                tail = qub[j, pl.ds(_H, 16)]
                sj = jnp.sum(prod) + ea0v[j] * tail[0] + ea1v[j] * tail[1]
                logit = jnp.where(lanes == j, sj, logit)
            logit = logit * 0.125
            e16 = jnp.exp(jnp.minimum(logit, 75.0))
            ebuf[...] = e16
            for j in range(16):
                ej = e16[j]
                t0 = ej * ea0v[j]
                t1 = ej * ea1v[j]
                for t in range(4):
                    vrow = kvb[j, pl.ds(_H + t * 16, 16)]
                    w0 = wevb[pl.ds(t * 16, 16)]
                    w1 = wevb[pl.ds(_H + t * 16, 16)]
                    vst[j, pl.ds(t * 16, 16)] = vrow * ej + w0 * t0 + w1 * t1
            pltpu.sync_copy(vst, accs.at[dst16], add=True)
            pltpu.sync_copy(ebuf, sss.at[dst16], add=True)
            return carry

        lax.fori_loop(0, nblk, blk, 0)
        plsc.subcore_barrier()

        def wbody(i, c):
            pltpu.sync_copy(accs.at[pl.ds(r0 + i * 16, 16)], vst)
            pltpu.sync_copy(vst, acc_o.at[cid, pl.ds(r0 + i * 16, 16)])
            pltpu.sync_copy(sss.at[pl.ds(r0 + i * 16, 16)], ebuf)
            pltpu.sync_copy(ebuf, ss_o.at[cid, pl.ds(r0 + i * 16, 16)])
            return c

        lax.fori_loop(0, n16, wbody, 0)
        for tail, nf, pred in ((tail_nom, n16_nom, sid != 15),
                               (tail_last, n16_last, sid == 15)):
            if tail:
                @pl.when(pred)
                def _(tail=tail, nf=nf):
                    o = r0 + nf * 16
                    pltpu.sync_copy(accs.at[pl.ds(o, tail)],
                                    vst.at[pl.ds(0, tail)])
                    pltpu.sync_copy(vst.at[pl.ds(0, tail)],
                                    acc_o.at[cid, pl.ds(o, tail)])
                    pltpu.sync_copy(sss.at[pl.ds(o, tail)],
                                    ebuf.at[pl.ds(0, tail)])
                    pltpu.sync_copy(ebuf.at[pl.ds(0, tail)],
                                    ss_o.at[cid, pl.ds(o, tail)])

    return k


def _fused_weights(params, layer, t, ind):
    blocks = []
    for (s, d) in _DST_ETS[t]:
        p = params[f"l{layer}_{s}__{d}"]
        qu = jnp.concatenate(
            [p["Wq"], p["Wq"] @ p["Wek"].T,
             jnp.zeros((ind, _QU_W - _H - 2), jnp.float32)], axis=1)
        blocks.append(qu)
    for (s, d) in _SRC_ETS[t]:
        p = params[f"l{layer}_{s}__{d}"]
        blocks.append(jnp.concatenate([p["Wk"], p["Wv"]], axis=1))
    ws = None
    for (s, d) in _DST_ETS[t]:
        w = params[f"l{layer}_{s}__{d}"]["Ws"]
        ws = w if ws is None else ws + w
    blocks.append(ws)
    w = jnp.concatenate(blocks, axis=1)
    _, total = _LAYOUT[t]
    if w.shape[1] < total:
        w = jnp.pad(w, ((0, 0), (0, total - w.shape[1])))
    if ind == 2:
        w = jnp.pad(w, ((0, 6), (0, 0)))
    return w


def kernel(x_SB, x_PQ, x_PV, x_NB, ei_SB_PQ, ei_PQ_SB, ei_PV_PQ, ei_PQ_PV,
           ei_NB_PQ, ei_PQ_NB, ei_PQ_PQ, ea_SB_PQ, ea_PQ_SB, ea_PV_PQ,
           ea_PQ_PV, ea_NB_PQ, ea_PQ_NB, ea_PQ_PQ, params):
    xs = {"SB": x_SB, "PQ": x_PQ, "PV": x_PV, "NB": x_NB}
    eis = {("SB", "PQ"): ei_SB_PQ, ("PQ", "SB"): ei_PQ_SB,
           ("PV", "PQ"): ei_PV_PQ, ("PQ", "PV"): ei_PQ_PV,
           ("NB", "PQ"): ei_NB_PQ, ("PQ", "NB"): ei_PQ_NB,
           ("PQ", "PQ"): ei_PQ_PQ}
    eas = {("SB", "PQ"): ea_SB_PQ, ("PQ", "SB"): ea_PQ_SB,
           ("PV", "PQ"): ea_PV_PQ, ("PQ", "PV"): ea_PQ_PV,
           ("NB", "PQ"): ea_NB_PQ, ("PQ", "NB"): ea_PQ_NB,
           ("PQ", "PQ"): ea_PQ_PQ}

    edge = {}
    for et in _ETS:
        s, d = et
        e, epad = _E[et], _EPAD[et]
        src = eis[et][0]
        dst = eis[et][1]
        ea0 = eas[et][:, 0]
        ea1 = eas[et][:, 1]
        pad = epad - e
        if pad:
            src = jnp.concatenate([src, jnp.zeros((pad,), jnp.int32)])
            dst = jnp.concatenate([dst, jnp.full((pad,), _N[d], jnp.int32)])
            ea0 = jnp.concatenate([ea0, jnp.zeros((pad,), jnp.float32)])
            ea1 = jnp.concatenate([ea1, jnp.zeros((pad,), jnp.float32)])
        edge[et] = (src, dst, ea0, ea1)

    h = {}
    for t in _TYPES:
        x0 = xs[t][:, 2:4]
        h[t] = jnp.pad(x0, ((0, _NPAD[t] - _N[t]), (0, 6)))

    for layer in range(2):
        ind = 2 if layer == 0 else _H
        proj = {}
        for t in _TYPES:
            w = _fused_weights(params, layer, t, ind)
            proj[t] = _mm(h[t], w)
        conv = {}
        for et in _ETS:
            s, d = et
            offs_d, _ = _LAYOUT[d]
            offs_s, _ = _LAYOUT[s]
            qu = proj[d][:, offs_d[("qu", et)]:offs_d[("qu", et)] + _QU_W]
            kv = proj[s][:, offs_s[("kv", et)]:offs_s[("kv", et)] + _KV_W]
            p = params[f"l{layer}_{s}__{d}"]
            wev = jnp.concatenate([p["Wev"][0], p["Wev"][1]])
            src, dst, ea0, ea1 = edge[et]
            acc, ssum = _edge_kernel(_EPAD[et], _NPAD[d], _NACC[d])(
                qu, kv, src, dst, ea0, ea1, wev)
            conv[et] = (acc, ssum)
        newh = {}
        for t in _TYPES:
            offs, _ = _LAYOUT[t]
            root = proj[t][:, offs["root"]:offs["root"] + _H]
            accs = [conv[et][0] for et in _DST_ETS[t]]
            ssums = [conv[et][1] for et in _DST_ETS[t]]
            newh[t] = _merge(accs, ssums, root, _N[t], _NPAD[t])
        h = newh

    return jnp.concatenate([h[t][:_N[t]] for t in _TYPES], axis=0)


# speedup vs baseline: 5.8238x; 1.0632x over previous
import functools

import jax
import jax.numpy as jnp
from jax import lax
from jax.experimental import pallas as pl
from jax.experimental.pallas import tpu as pltpu
from jax.experimental.pallas import tpu_sc as plsc

_H = 64
_TYPES = ("SB", "PQ", "PV", "NB")
_N = {"SB": 500, "PQ": 30000, "PV": 15000, "NB": 4500}
_ETS = (("SB", "PQ"), ("PQ", "SB"), ("PV", "PQ"), ("PQ", "PV"),
        ("NB", "PQ"), ("PQ", "NB"), ("PQ", "PQ"))
_E = {("SB", "PQ"): 50000, ("PQ", "SB"): 50000, ("PV", "PQ"): 150000,
      ("PQ", "PV"): 150000, ("NB", "PQ"): 50000, ("PQ", "NB"): 50000,
      ("PQ", "PQ"): 300000}


def _rup(x, m):
    return (x + m - 1) // m * m


_NPAD = {t: _rup(_N[t] + 1, 256) for t in _TYPES}
_NACC = {t: _rup(_N[t] + 1, 8) for t in _TYPES}
_EPAD = {et: _rup(_E[et], 2048) for et in _ETS}
_DST_ETS = {t: tuple(et for et in _ETS if et[1] == t) for t in _TYPES}
_SRC_ETS = {t: tuple(et for et in _ETS if et[0] == t) for t in _TYPES}

_QU_W = 80
_KV_W = 128


def _col_layout(t):
    offs = {}
    c = 0
    for et in _DST_ETS[t]:
        offs[("qu", et)] = c
        c += _QU_W
    for et in _SRC_ETS[t]:
        offs[("kv", et)] = c
        c += _KV_W
    offs["root"] = c
    c += _H
    return offs, _rup(c, 128)


_LAYOUT = {t: _col_layout(t) for t in _TYPES}


def _mm_body(x_ref, w_ref, o_ref):
    o_ref[...] = jnp.dot(x_ref[...], w_ref[...],
                         preferred_element_type=jnp.float32)


def _mm(x, w):
    m, k = x.shape
    _, n = w.shape
    bm = 256
    return pl.pallas_call(
        _mm_body,
        grid=(m // bm,),
        in_specs=[pl.BlockSpec((bm, k), lambda i: (i, 0)),
                  pl.BlockSpec((k, n), lambda i: (0, 0))],
        out_specs=pl.BlockSpec((bm, n), lambda i: (i, 0)),
        out_shape=jax.ShapeDtypeStruct((m, n), jnp.float32),
    )(x, w)


def _merge_body(n_real, n_ets, bm, *refs):
    acc_refs = refs[:n_ets]
    ss_refs = refs[n_ets:2 * n_ets]
    root_ref = refs[2 * n_ets]
    o_ref = refs[2 * n_ets + 1]
    val = root_ref[...]
    for a_r, s_r in zip(acc_refs, ss_refs):
        a = a_r[0] + a_r[1]
        s = s_r[0] + s_r[1]
        val = val + a / (s[:, None] + 1e-16)
    neg = jnp.exp(jnp.minimum(val, 0.0)) - 1.0
    val = jnp.where(val > 0, val, neg)
    i = pl.program_id(0)
    rows = i * bm + lax.broadcasted_iota(jnp.int32, (bm, _H), 0)
    o_ref[...] = jnp.where(rows < n_real, val, 0.0)


def _merge(accs, ssums, root, n_real, npad):
    n_ets = len(accs)
    bm = 256
    in_specs = (
        [pl.BlockSpec((2, bm, _H), lambda i: (0, i, 0)) for _ in accs]
        + [pl.BlockSpec((2, bm), lambda i: (0, i)) for _ in ssums]
        + [pl.BlockSpec((bm, _H), lambda i: (i, 0))]
    )
    return pl.pallas_call(
        functools.partial(_merge_body, n_real, n_ets, bm),
        grid=(npad // bm,),
        in_specs=in_specs,
        out_specs=pl.BlockSpec((bm, _H), lambda i: (i, 0)),
        out_shape=jax.ShapeDtypeStruct((npad, _H), jnp.float32),
    )(*accs, *ssums, root)


@functools.lru_cache(None)
def _edge_kernel(epad, npad, nacc):
    chunk = epad // 32
    bsz = 32
    nblk = chunk // bsz
    rpt = _rup((nacc + 15) // 16, 8)
    last = nacc - 15 * rpt
    n16_nom, tail_nom = rpt // 16, rpt % 16
    n16_last, tail_last = last // 16, last % 16
    mesh = plsc.VectorSubcoreMesh(core_axis_name="c", subcore_axis_name="s")

    @functools.partial(
        pl.kernel,
        mesh=mesh,
        compiler_params=pltpu.CompilerParams(needs_layout_passes=False,
                                             use_tc_tiling_on_sc=False),
        out_type=(jax.ShapeDtypeStruct((2, npad, _H), jnp.float32),
                  jax.ShapeDtypeStruct((2, npad), jnp.float32)),
        scratch_types=[
            pltpu.VMEM((bsz,), jnp.int32),
            pltpu.VMEM((bsz,), jnp.int32),
            pltpu.VMEM((bsz,), jnp.float32),
            pltpu.VMEM((bsz,), jnp.float32),
            pltpu.VMEM((bsz, _QU_W), jnp.float32),
            pltpu.VMEM((bsz, _KV_W), jnp.float32),
            pltpu.VMEM((bsz, _H), jnp.float32),
            pltpu.VMEM((bsz,), jnp.float32),
            pltpu.VMEM((128,), jnp.float32),
            pltpu.VMEM_SHARED((nacc, _H), jnp.float32),
            pltpu.VMEM_SHARED((nacc,), jnp.float32),
            pltpu.SemaphoreType.DMA,
            pltpu.SemaphoreType.DMA,
            pltpu.SemaphoreType.DMA,
            pltpu.SemaphoreType.DMA,
        ],
    )
    def k(qu_h, kv_h, src_h, dst_h, ea0_h, ea1_h, wev_h,
          acc_o, ss_o, srcb, dstb, ea0b, ea1b, qub, kvb, vst, ebuf, wevb,
          accs, sss, sem1, sem2, sem3, sem4):
        cid = lax.axis_index("c")
        sid = lax.axis_index("s")
        wid = sid * 2 + cid
        base = wid * chunk
        pltpu.sync_copy(wev_h, wevb)
        r0 = sid * rpt
        n16 = jnp.where(sid == 15, n16_last, n16_nom)
        z16 = jnp.zeros((16,), jnp.float32)
        for r in range(16):
            for t in range(4):
                vst[r, pl.ds(t * 16, 16)] = z16
        ebuf[pl.ds(0, 16)] = z16

        def zbody(i, c):
            pltpu.sync_copy(vst.at[pl.ds(0, 16)],
                            accs.at[pl.ds(r0 + i * 16, 16)])
            pltpu.sync_copy(ebuf.at[pl.ds(0, 16)],
                            sss.at[pl.ds(r0 + i * 16, 16)])
            return c

        lax.fori_loop(0, n16, zbody, 0)
        for tail, nf, pred in ((tail_nom, n16_nom, sid != 15),
                               (tail_last, n16_last, sid == 15)):
            if tail:
                @pl.when(pred)
                def _(tail=tail, nf=nf):
                    o = r0 + nf * 16
                    pltpu.sync_copy(vst.at[pl.ds(0, tail)],
                                    accs.at[pl.ds(o, tail)])
                    pltpu.sync_copy(ebuf.at[pl.ds(0, tail)],
                                    sss.at[pl.ds(o, tail)])
        plsc.subcore_barrier()

        lanes = lax.iota(jnp.int32, 16)

        def blk(b, carry):
            boff = base + b * bsz
            st = [pltpu.async_copy(src_h.at[pl.ds(boff, bsz)], srcb, sem3),
                  pltpu.async_copy(dst_h.at[pl.ds(boff, bsz)], dstb, sem3),
                  pltpu.async_copy(ea0_h.at[pl.ds(boff, bsz)], ea0b, sem3),
                  pltpu.async_copy(ea1_h.at[pl.ds(boff, bsz)], ea1b, sem3)]
            for cp in st:
                cp.wait()
            cp1 = pltpu.async_copy(qu_h.at[dstb], qub, sem1)
            cp2 = pltpu.async_copy(kv_h.at[srcb], kvb, sem2)
            cp1.wait()
            cp2.wait()
            def grp(g, c2):
                off = g * 16
                ea0v = ea0b[pl.ds(off, 16)]
                ea1v = ea1b[pl.ds(off, 16)]
                logit = jnp.zeros((16,), jnp.float32)
                for j in range(16):
                    r = off + j
                    prod = qub[r, pl.ds(0, 16)] * kvb[r, pl.ds(0, 16)]
                    for t in range(1, 4):
                        prod = prod + (qub[r, pl.ds(t * 16, 16)]
                                       * kvb[r, pl.ds(t * 16, 16)])
                    tail = qub[r, pl.ds(_H, 16)]
                    sj = jnp.sum(prod) + ea0v[j] * tail[0] + ea1v[j] * tail[1]
                    logit = jnp.where(lanes == j, sj, logit)
                logit = logit * 0.125
                e16 = jnp.exp(jnp.minimum(logit, 75.0))
                ebuf[pl.ds(off, 16)] = e16
                for j in range(16):
                    r = off + j
                    ej = e16[j]
                    t0 = ej * ea0v[j]
                    t1 = ej * ea1v[j]
                    for t in range(4):
                        vrow = kvb[r, pl.ds(_H + t * 16, 16)]
                        w0 = wevb[pl.ds(t * 16, 16)]
                        w1 = wevb[pl.ds(_H + t * 16, 16)]
                        vst[r, pl.ds(t * 16, 16)] = (vrow * ej + w0 * t0
                                                     + w1 * t1)
                return c2

            lax.fori_loop(0, bsz // 16, grp, 0)
            sc1 = pltpu.async_copy(vst, accs.at[dstb], sem4, add=True)
            sc2 = pltpu.async_copy(ebuf, sss.at[dstb], sem4, add=True)
            sc1.wait()
            sc2.wait()
            return carry

        lax.fori_loop(0, nblk, blk, 0)
        plsc.subcore_barrier()

        def wbody(i, c):
            pltpu.sync_copy(accs.at[pl.ds(r0 + i * 16, 16)],
                            vst.at[pl.ds(0, 16)])
            pltpu.sync_copy(vst.at[pl.ds(0, 16)],
                            acc_o.at[cid, pl.ds(r0 + i * 16, 16)])
            pltpu.sync_copy(sss.at[pl.ds(r0 + i * 16, 16)],
                            ebuf.at[pl.ds(0, 16)])
            pltpu.sync_copy(ebuf.at[pl.ds(0, 16)],
                            ss_o.at[cid, pl.ds(r0 + i * 16, 16)])
            return c

        lax.fori_loop(0, n16, wbody, 0)
        for tail, nf, pred in ((tail_nom, n16_nom, sid != 15),
                               (tail_last, n16_last, sid == 15)):
            if tail:
                @pl.when(pred)
                def _(tail=tail, nf=nf):
                    o = r0 + nf * 16
                    pltpu.sync_copy(accs.at[pl.ds(o, tail)],
                                    vst.at[pl.ds(0, tail)])
                    pltpu.sync_copy(vst.at[pl.ds(0, tail)],
                                    acc_o.at[cid, pl.ds(o, tail)])
                    pltpu.sync_copy(sss.at[pl.ds(o, tail)],
                                    ebuf.at[pl.ds(0, tail)])
                    pltpu.sync_copy(ebuf.at[pl.ds(0, tail)],
                                    ss_o.at[cid, pl.ds(o, tail)])

    return k


def _fused_weights(params, layer, t, ind):
    blocks = []
    for (s, d) in _DST_ETS[t]:
        p = params[f"l{layer}_{s}__{d}"]
        qu = jnp.concatenate(
            [p["Wq"], p["Wq"] @ p["Wek"].T,
             jnp.zeros((ind, _QU_W - _H - 2), jnp.float32)], axis=1)
        blocks.append(qu)
    for (s, d) in _SRC_ETS[t]:
        p = params[f"l{layer}_{s}__{d}"]
        blocks.append(jnp.concatenate([p["Wk"], p["Wv"]], axis=1))
    ws = None
    for (s, d) in _DST_ETS[t]:
        w = params[f"l{layer}_{s}__{d}"]["Ws"]
        ws = w if ws is None else ws + w
    blocks.append(ws)
    w = jnp.concatenate(blocks, axis=1)
    _, total = _LAYOUT[t]
    if w.shape[1] < total:
        w = jnp.pad(w, ((0, 0), (0, total - w.shape[1])))
    if ind == 2:
        w = jnp.pad(w, ((0, 6), (0, 0)))
    return w


def kernel(x_SB, x_PQ, x_PV, x_NB, ei_SB_PQ, ei_PQ_SB, ei_PV_PQ, ei_PQ_PV,
           ei_NB_PQ, ei_PQ_NB, ei_PQ_PQ, ea_SB_PQ, ea_PQ_SB, ea_PV_PQ,
           ea_PQ_PV, ea_NB_PQ, ea_PQ_NB, ea_PQ_PQ, params):
    xs = {"SB": x_SB, "PQ": x_PQ, "PV": x_PV, "NB": x_NB}
    eis = {("SB", "PQ"): ei_SB_PQ, ("PQ", "SB"): ei_PQ_SB,
           ("PV", "PQ"): ei_PV_PQ, ("PQ", "PV"): ei_PQ_PV,
           ("NB", "PQ"): ei_NB_PQ, ("PQ", "NB"): ei_PQ_NB,
           ("PQ", "PQ"): ei_PQ_PQ}
    eas = {("SB", "PQ"): ea_SB_PQ, ("PQ", "SB"): ea_PQ_SB,
           ("PV", "PQ"): ea_PV_PQ, ("PQ", "PV"): ea_PQ_PV,
           ("NB", "PQ"): ea_NB_PQ, ("PQ", "NB"): ea_PQ_NB,
           ("PQ", "PQ"): ea_PQ_PQ}

    edge = {}
    for et in _ETS:
        s, d = et
        e, epad = _E[et], _EPAD[et]
        src = eis[et][0]
        dst = eis[et][1]
        ea0 = eas[et][:, 0]
        ea1 = eas[et][:, 1]
        pad = epad - e
        if pad:
            src = jnp.concatenate([src, jnp.zeros((pad,), jnp.int32)])
            dst = jnp.concatenate([dst, jnp.full((pad,), _N[d], jnp.int32)])
            ea0 = jnp.concatenate([ea0, jnp.zeros((pad,), jnp.float32)])
            ea1 = jnp.concatenate([ea1, jnp.zeros((pad,), jnp.float32)])
        edge[et] = (src, dst, ea0, ea1)

    h = {}
    for t in _TYPES:
        x0 = xs[t][:, 2:4]
        h[t] = jnp.pad(x0, ((0, _NPAD[t] - _N[t]), (0, 6)))

    for layer in range(2):
        ind = 2 if layer == 0 else _H
        proj = {}
        for t in _TYPES:
            w = _fused_weights(params, layer, t, ind)
            proj[t] = _mm(h[t], w)
        conv = {}
        for et in _ETS:
            s, d = et
            offs_d, _ = _LAYOUT[d]
            offs_s, _ = _LAYOUT[s]
            qu = proj[d][:, offs_d[("qu", et)]:offs_d[("qu", et)] + _QU_W]
            kv = proj[s][:, offs_s[("kv", et)]:offs_s[("kv", et)] + _KV_W]
            p = params[f"l{layer}_{s}__{d}"]
            wev = jnp.concatenate([p["Wev"][0], p["Wev"][1]])
            src, dst, ea0, ea1 = edge[et]
            acc, ssum = _edge_kernel(_EPAD[et], _NPAD[d], _NACC[d])(
                qu, kv, src, dst, ea0, ea1, wev)
            conv[et] = (acc, ssum)
        newh = {}
        for t in _TYPES:
            offs, _ = _LAYOUT[t]
            root = proj[t][:, offs["root"]:offs["root"] + _H]
            accs = [conv[et][0] for et in _DST_ETS[t]]
            ssums = [conv[et][1] for et in _DST_ETS[t]]
            newh[t] = _merge(accs, ssums, root, _N[t], _NPAD[t])
        h = newh

    return jnp.concatenate([h[t][:_N[t]] for t in _TYPES], axis=0)


# speedup vs baseline: 6.1046x; 1.0482x over previous
import functools

import jax
import jax.numpy as jnp
from jax import lax
from jax.experimental import pallas as pl
from jax.experimental.pallas import tpu as pltpu
from jax.experimental.pallas import tpu_sc as plsc

_H = 64
_TYPES = ("SB", "PQ", "PV", "NB")
_N = {"SB": 500, "PQ": 30000, "PV": 15000, "NB": 4500}
_ETS = (("SB", "PQ"), ("PQ", "SB"), ("PV", "PQ"), ("PQ", "PV"),
        ("NB", "PQ"), ("PQ", "NB"), ("PQ", "PQ"))
_E = {("SB", "PQ"): 50000, ("PQ", "SB"): 50000, ("PV", "PQ"): 150000,
      ("PQ", "PV"): 150000, ("NB", "PQ"): 50000, ("PQ", "NB"): 50000,
      ("PQ", "PQ"): 300000}


def _rup(x, m):
    return (x + m - 1) // m * m


_NPAD = {t: _rup(_N[t] + 1, 256) for t in _TYPES}
_NACC = {t: _rup(_N[t] + 1, 8) for t in _TYPES}
_EPAD = {et: _rup(_E[et], 2048) for et in _ETS}
_DST_ETS = {t: tuple(et for et in _ETS if et[1] == t) for t in _TYPES}
_SRC_ETS = {t: tuple(et for et in _ETS if et[0] == t) for t in _TYPES}

_QU_W = 80
_KV_W = 128


def _col_layout(t):
    offs = {}
    c = 0
    for et in _DST_ETS[t]:
        offs[("qu", et)] = c
        c += _QU_W
    for et in _SRC_ETS[t]:
        offs[("kv", et)] = c
        c += _KV_W
    offs["root"] = c
    c += _H
    return offs, _rup(c, 128)


_LAYOUT = {t: _col_layout(t) for t in _TYPES}


def _mm_body(x_ref, w_ref, o_ref):
    o_ref[...] = jnp.dot(x_ref[...], w_ref[...],
                         preferred_element_type=jnp.float32)


def _mm(x, w):
    m, k = x.shape
    _, n = w.shape
    bm = 256
    return pl.pallas_call(
        _mm_body,
        grid=(m // bm,),
        in_specs=[pl.BlockSpec((bm, k), lambda i: (i, 0)),
                  pl.BlockSpec((k, n), lambda i: (0, 0))],
        out_specs=pl.BlockSpec((bm, n), lambda i: (i, 0)),
        out_shape=jax.ShapeDtypeStruct((m, n), jnp.float32),
    )(x, w)


def _merge_body(n_real, n_ets, bm, *refs):
    acc_refs = refs[:n_ets]
    ss_refs = refs[n_ets:2 * n_ets]
    root_ref = refs[2 * n_ets]
    o_ref = refs[2 * n_ets + 1]
    val = root_ref[...]
    for a_r, s_r in zip(acc_refs, ss_refs):
        a = a_r[0] + a_r[1]
        s = s_r[0] + s_r[1]
        val = val + a / (s[:, None] + 1e-16)
    neg = jnp.exp(jnp.minimum(val, 0.0)) - 1.0
    val = jnp.where(val > 0, val, neg)
    i = pl.program_id(0)
    rows = i * bm + lax.broadcasted_iota(jnp.int32, (bm, _H), 0)
    o_ref[...] = jnp.where(rows < n_real, val, 0.0)


def _merge(accs, ssums, root, n_real, npad):
    n_ets = len(accs)
    bm = 256
    in_specs = (
        [pl.BlockSpec((2, bm, _H), lambda i: (0, i, 0)) for _ in accs]
        + [pl.BlockSpec((2, bm), lambda i: (0, i)) for _ in ssums]
        + [pl.BlockSpec((bm, _H), lambda i: (i, 0))]
    )
    return pl.pallas_call(
        functools.partial(_merge_body, n_real, n_ets, bm),
        grid=(npad // bm,),
        in_specs=in_specs,
        out_specs=pl.BlockSpec((bm, _H), lambda i: (i, 0)),
        out_shape=jax.ShapeDtypeStruct((npad, _H), jnp.float32),
    )(*accs, *ssums, root)


@functools.lru_cache(None)
def _edge_kernel(epad, npad, nacc):
    chunk = epad // 32
    bsz = 32
    nblk = chunk // bsz
    rpt = _rup((nacc + 15) // 16, 8)
    last = nacc - 15 * rpt
    n32_nom, tail_nom = rpt // 32, rpt % 32
    n32_last, tail_last = last // 32, last % 32
    mesh = plsc.VectorSubcoreMesh(core_axis_name="c", subcore_axis_name="s")

    @functools.partial(
        pl.kernel,
        mesh=mesh,
        compiler_params=pltpu.CompilerParams(needs_layout_passes=False,
                                             use_tc_tiling_on_sc=False),
        out_type=(jax.ShapeDtypeStruct((2, npad, _H), jnp.float32),
                  jax.ShapeDtypeStruct((2, npad), jnp.float32)),
        scratch_types=[
            pltpu.VMEM((bsz,), jnp.int32),
            pltpu.VMEM((bsz,), jnp.int32),
            pltpu.VMEM((bsz,), jnp.float32),
            pltpu.VMEM((bsz,), jnp.float32),
            pltpu.VMEM((bsz, _QU_W), jnp.float32),
            pltpu.VMEM((bsz, _KV_W), jnp.float32),
            pltpu.VMEM((bsz, _H), jnp.float32),
            pltpu.VMEM((bsz,), jnp.float32),
            pltpu.VMEM((128,), jnp.float32),
            pltpu.VMEM_SHARED((nacc, _H), jnp.float32),
            pltpu.VMEM_SHARED((nacc,), jnp.float32),
            pltpu.SemaphoreType.DMA,
            pltpu.SemaphoreType.DMA,
            pltpu.SemaphoreType.DMA,
            pltpu.SemaphoreType.DMA,
        ],
    )
    def k(qu_h, kv_h, src_h, dst_h, ea0_h, ea1_h, wev_h,
          acc_o, ss_o, srcb, dstb, ea0b, ea1b, qub, kvb, vst, ebuf, wevb,
          accs, sss, sem1, sem2, sem3, sem4):
        cid = lax.axis_index("c")
        sid = lax.axis_index("s")
        wid = sid * 2 + cid
        base = wid * chunk
        pltpu.sync_copy(wev_h, wevb)
        r0 = sid * rpt
        n32 = jnp.where(sid == 15, n32_last, n32_nom)
        z16 = jnp.zeros((16,), jnp.float32)
        for r in range(32):
            for t in range(4):
                vst[r, pl.ds(t * 16, 16)] = z16
        for t in range(2):
            ebuf[pl.ds(t * 16, 16)] = z16

        def zbody(i, c):
            c1 = pltpu.async_copy(vst.at[pl.ds(0, 32)],
                                  accs.at[pl.ds(r0 + i * 32, 32)], sem1)
            c2 = pltpu.async_copy(ebuf.at[pl.ds(0, 32)],
                                  sss.at[pl.ds(r0 + i * 32, 32)], sem2)
            c1.wait()
            c2.wait()
            return c

        lax.fori_loop(0, n32, zbody, 0)
        for tail, nf, pred in ((tail_nom, n32_nom, sid != 15),
                               (tail_last, n32_last, sid == 15)):
            if tail:
                @pl.when(pred)
                def _(tail=tail, nf=nf):
                    o = r0 + nf * 32
                    pltpu.sync_copy(vst.at[pl.ds(0, tail)],
                                    accs.at[pl.ds(o, tail)])
                    pltpu.sync_copy(ebuf.at[pl.ds(0, tail)],
                                    sss.at[pl.ds(o, tail)])
        plsc.subcore_barrier()

        lanes = lax.iota(jnp.int32, 16)

        def blk(b, carry):
            boff = base + b * bsz
            st = [pltpu.async_copy(src_h.at[pl.ds(boff, bsz)], srcb, sem3),
                  pltpu.async_copy(dst_h.at[pl.ds(boff, bsz)], dstb, sem3),
                  pltpu.async_copy(ea0_h.at[pl.ds(boff, bsz)], ea0b, sem3),
                  pltpu.async_copy(ea1_h.at[pl.ds(boff, bsz)], ea1b, sem3)]
            for cp in st:
                cp.wait()
            cp1 = pltpu.async_copy(qu_h.at[dstb], qub, sem1)
            cp2 = pltpu.async_copy(kv_h.at[srcb], kvb, sem2)
            cp1.wait()
            cp2.wait()
            def grp(g, c2):
                off = g * 16
                ea0v = ea0b[pl.ds(off, 16)]
                ea1v = ea1b[pl.ds(off, 16)]
                logit = jnp.zeros((16,), jnp.float32)
                for j in range(16):
                    r = off + j
                    prod = qub[r, pl.ds(0, 16)] * kvb[r, pl.ds(0, 16)]
                    for t in range(1, 4):
                        prod = prod + (qub[r, pl.ds(t * 16, 16)]
                                       * kvb[r, pl.ds(t * 16, 16)])
                    tail = qub[r, pl.ds(_H, 16)]
                    sj = jnp.sum(prod) + ea0v[j] * tail[0] + ea1v[j] * tail[1]
                    logit = jnp.where(lanes == j, sj, logit)
                logit = logit * 0.125
                e16 = jnp.exp(jnp.minimum(logit, 75.0))
                ebuf[pl.ds(off, 16)] = e16
                for j in range(16):
                    r = off + j
                    ej = e16[j]
                    t0 = ej * ea0v[j]
                    t1 = ej * ea1v[j]
                    for t in range(4):
                        vrow = kvb[r, pl.ds(_H + t * 16, 16)]
                        w0 = wevb[pl.ds(t * 16, 16)]
                        w1 = wevb[pl.ds(_H + t * 16, 16)]
                        vst[r, pl.ds(t * 16, 16)] = (vrow * ej + w0 * t0
                                                     + w1 * t1)
                return c2

            lax.fori_loop(0, bsz // 16, grp, 0)
            sc1 = pltpu.async_copy(vst, accs.at[dstb], sem4, add=True)
            sc2 = pltpu.async_copy(ebuf, sss.at[dstb], sem4, add=True)
            sc1.wait()
            sc2.wait()
            return carry

        lax.fori_loop(0, nblk, blk, 0)
        plsc.subcore_barrier()

        def wbody(i, c):
            c1 = pltpu.async_copy(accs.at[pl.ds(r0 + i * 32, 32)],
                                  vst.at[pl.ds(0, 32)], sem1)
            c2 = pltpu.async_copy(sss.at[pl.ds(r0 + i * 32, 32)],
                                  ebuf.at[pl.ds(0, 32)], sem2)
            c1.wait()
            c2.wait()
            c3 = pltpu.async_copy(vst.at[pl.ds(0, 32)],
                                  acc_o.at[cid, pl.ds(r0 + i * 32, 32)], sem1)
            c4 = pltpu.async_copy(ebuf.at[pl.ds(0, 32)],
                                  ss_o.at[cid, pl.ds(r0 + i * 32, 32)], sem2)
            c3.wait()
            c4.wait()
            return c

        lax.fori_loop(0, n32, wbody, 0)
        for tail, nf, pred in ((tail_nom, n32_nom, sid != 15),
                               (tail_last, n32_last, sid == 15)):
            if tail:
                @pl.when(pred)
                def _(tail=tail, nf=nf):
                    o = r0 + nf * 32
                    pltpu.sync_copy(accs.at[pl.ds(o, tail)],
                                    vst.at[pl.ds(0, tail)])
                    pltpu.sync_copy(vst.at[pl.ds(0, tail)],
                                    acc_o.at[cid, pl.ds(o, tail)])
                    pltpu.sync_copy(sss.at[pl.ds(o, tail)],
                                    ebuf.at[pl.ds(0, tail)])
                    pltpu.sync_copy(ebuf.at[pl.ds(0, tail)],
                                    ss_o.at[cid, pl.ds(o, tail)])

    return k


def _fused_weights(params, layer, t, ind):
    blocks = []
    for (s, d) in _DST_ETS[t]:
        p = params[f"l{layer}_{s}__{d}"]
        qu = jnp.concatenate(
            [p["Wq"], p["Wq"] @ p["Wek"].T,
             jnp.zeros((ind, _QU_W - _H - 2), jnp.float32)], axis=1)
        blocks.append(qu)
    for (s, d) in _SRC_ETS[t]:
        p = params[f"l{layer}_{s}__{d}"]
        blocks.append(jnp.concatenate([p["Wk"], p["Wv"]], axis=1))
    ws = None
    for (s, d) in _DST_ETS[t]:
        w = params[f"l{layer}_{s}__{d}"]["Ws"]
        ws = w if ws is None else ws + w
    blocks.append(ws)
    w = jnp.concatenate(blocks, axis=1)
    _, total = _LAYOUT[t]
    if w.shape[1] < total:
        w = jnp.pad(w, ((0, 0), (0, total - w.shape[1])))
    if ind == 2:
        w = jnp.pad(w, ((0, 6), (0, 0)))
    return w


def kernel(x_SB, x_PQ, x_PV, x_NB, ei_SB_PQ, ei_PQ_SB, ei_PV_PQ, ei_PQ_PV,
           ei_NB_PQ, ei_PQ_NB, ei_PQ_PQ, ea_SB_PQ, ea_PQ_SB, ea_PV_PQ,
           ea_PQ_PV, ea_NB_PQ, ea_PQ_NB, ea_PQ_PQ, params):
    xs = {"SB": x_SB, "PQ": x_PQ, "PV": x_PV, "NB": x_NB}
    eis = {("SB", "PQ"): ei_SB_PQ, ("PQ", "SB"): ei_PQ_SB,
           ("PV", "PQ"): ei_PV_PQ, ("PQ", "PV"): ei_PQ_PV,
           ("NB", "PQ"): ei_NB_PQ, ("PQ", "NB"): ei_PQ_NB,
           ("PQ", "PQ"): ei_PQ_PQ}
    eas = {("SB", "PQ"): ea_SB_PQ, ("PQ", "SB"): ea_PQ_SB,
           ("PV", "PQ"): ea_PV_PQ, ("PQ", "PV"): ea_PQ_PV,
           ("NB", "PQ"): ea_NB_PQ, ("PQ", "NB"): ea_PQ_NB,
           ("PQ", "PQ"): ea_PQ_PQ}

    edge = {}
    for et in _ETS:
        s, d = et
        e, epad = _E[et], _EPAD[et]
        src = eis[et][0]
        dst = eis[et][1]
        ea0 = eas[et][:, 0]
        ea1 = eas[et][:, 1]
        pad = epad - e
        if pad:
            src = jnp.concatenate([src, jnp.zeros((pad,), jnp.int32)])
            dst = jnp.concatenate([dst, jnp.full((pad,), _N[d], jnp.int32)])
            ea0 = jnp.concatenate([ea0, jnp.zeros((pad,), jnp.float32)])
            ea1 = jnp.concatenate([ea1, jnp.zeros((pad,), jnp.float32)])
        edge[et] = (src, dst, ea0, ea1)

    h = {}
    for t in _TYPES:
        x0 = xs[t][:, 2:4]
        h[t] = jnp.pad(x0, ((0, _NPAD[t] - _N[t]), (0, 6)))

    for layer in range(2):
        ind = 2 if layer == 0 else _H
        proj = {}
        for t in _TYPES:
            w = _fused_weights(params, layer, t, ind)
            proj[t] = _mm(h[t], w)
        conv = {}
        for et in _ETS:
            s, d = et
            offs_d, _ = _LAYOUT[d]
            offs_s, _ = _LAYOUT[s]
            qu = proj[d][:, offs_d[("qu", et)]:offs_d[("qu", et)] + _QU_W]
            kv = proj[s][:, offs_s[("kv", et)]:offs_s[("kv", et)] + _KV_W]
            p = params[f"l{layer}_{s}__{d}"]
            wev = jnp.concatenate([p["Wev"][0], p["Wev"][1]])
            src, dst, ea0, ea1 = edge[et]
            acc, ssum = _edge_kernel(_EPAD[et], _NPAD[d], _NACC[d])(
                qu, kv, src, dst, ea0, ea1, wev)
            conv[et] = (acc, ssum)
        newh = {}
        for t in _TYPES:
            offs, _ = _LAYOUT[t]
            root = proj[t][:, offs["root"]:offs["root"] + _H]
            accs = [conv[et][0] for et in _DST_ETS[t]]
            ssums = [conv[et][1] for et in _DST_ETS[t]]
            newh[t] = _merge(accs, ssums, root, _N[t], _NPAD[t])
        h = newh

    return jnp.concatenate([h[t][:_N[t]] for t in _TYPES], axis=0)
